# Initial kernel scaffold; baseline (speedup 1.0000x reference)
import jax
import jax.numpy as jnp
from jax.experimental import pallas as pl


def kernel(fixed_verts, moving_verts, mat, trans):
    raise NotImplementedError("write your pallas kernel here")


# speedup vs baseline: 1.0139x; 1.0139x over previous
import functools

import jax
import jax.numpy as jnp
from jax.experimental import pallas as pl
from jax.experimental.pallas import tpu as pltpu

N1 = 8192
N2 = 8192
BM = 256


def _chamfer_kernel(fixed_ref, movT_ref, mataug_ref, out_ref,
                    yT_scr, y2_scr, colmin_scr, rowsum_scr):
    i = pl.program_id(0)
    nsteps = pl.num_programs(0)

    @pl.when(i == 0)
    def _init():
        yT = jnp.dot(mataug_ref[...], movT_ref[...],
                     preferred_element_type=jnp.float32)
        yT_scr[0:3, :] = yT
        y2_scr[0:1, :] = jnp.sum(yT * yT, axis=0, keepdims=True)
        colmin_scr[...] = jnp.full_like(colmin_scr, jnp.inf)
        rowsum_scr[...] = jnp.zeros_like(rowsum_scr)

    xb = fixed_ref[...]
    x2 = jnp.sum(xb * xb, axis=1, keepdims=True)
    gram = jnp.dot(xb, yT_scr[0:3, :],
                   preferred_element_type=jnp.float32)
    d2 = x2 + y2_scr[0:1, :] - 2.0 * gram
    d2 = jnp.maximum(d2, 0.0)

    row_min = jnp.min(d2, axis=1)
    rowsum_scr[...] += jnp.sum(row_min).reshape(1, 1)
    col_min = jnp.min(d2, axis=0, keepdims=True)
    colmin_scr[0:1, :] = jnp.minimum(colmin_scr[0:1, :], col_min)

    @pl.when(i == nsteps - 1)
    def _fin():
        out_ref[...] = (rowsum_scr[...] / N1
                        + jnp.sum(colmin_scr[0:1, :]).reshape(1, 1) / N2)


@jax.jit
def _chamfer(fixed_verts, movT_aug, mat_aug):
    grid = N1 // BM
    out = pl.pallas_call(
        _chamfer_kernel,
        grid=(grid,),
        in_specs=[
            pl.BlockSpec((BM, 3), lambda i: (i, 0)),
            pl.BlockSpec((4, N2), lambda i: (0, 0)),
            pl.BlockSpec((3, 4), lambda i: (0, 0)),
        ],
        out_specs=pl.BlockSpec((1, 1), lambda i: (0, 0)),
        out_shape=jax.ShapeDtypeStruct((1, 1), jnp.float32),
        scratch_shapes=[
            pltpu.VMEM((8, N2), jnp.float32),
            pltpu.VMEM((1, N2), jnp.float32),
            pltpu.VMEM((1, N2), jnp.float32),
            pltpu.VMEM((1, 1), jnp.float32),
        ],
    )(fixed_verts, movT_aug, mat_aug)
    return out[0, 0]


def kernel(fixed_verts, moving_verts, mat, trans):
    movT_aug = jnp.concatenate(
        [moving_verts.T, jnp.ones((1, N2), jnp.float32)], axis=0)
    mat_aug = jnp.concatenate([mat[0].T, trans[0]], axis=1)
    return _chamfer(fixed_verts, movT_aug, mat_aug)


# speedup vs baseline: 1.6379x; 1.6155x over previous
import jax
import jax.numpy as jnp
from jax.experimental import pallas as pl
from jax.experimental.pallas import tpu as pltpu

N1 = 8192
N2 = 8192
BM = 512


def _chamfer_kernel(fixed_ref, movT_ref, mataug_ref, out_ref,
                    rhs_scr, colmin_scr, rowsum_scr):
    i = pl.program_id(0)
    nsteps = pl.num_programs(0)

    @pl.when(i == 0)
    def _init():
        yT = jnp.dot(mataug_ref[...], movT_ref[...],
                     preferred_element_type=jnp.float32)
        rhs_scr[0:3, :] = yT
        rhs_scr[3:4, :] = jnp.sum(yT * yT, axis=0, keepdims=True)
        rhs_scr[4:5, :] = jnp.ones((1, N2), jnp.float32)
        colmin_scr[...] = jnp.full_like(colmin_scr, jnp.inf)
        rowsum_scr[...] = jnp.zeros_like(rowsum_scr)

    xb = fixed_ref[...]
    x2 = jnp.sum(xb * xb, axis=1, keepdims=True)
    lhs = jnp.concatenate(
        [xb * -2.0, jnp.ones((BM, 1), jnp.float32), x2], axis=1)
    d2 = jnp.dot(lhs, rhs_scr[0:5, :],
                 preferred_element_type=jnp.float32)

    row_min = jnp.maximum(jnp.min(d2, axis=1), 0.0)
    rowsum_scr[...] += jnp.sum(row_min).reshape(1, 1)
    col_min = jnp.min(d2, axis=0, keepdims=True)
    colmin_scr[...] = jnp.minimum(colmin_scr[...], col_min)

    @pl.when(i == nsteps - 1)
    def _fin():
        col_sum = jnp.sum(jnp.maximum(colmin_scr[...], 0.0))
        out_ref[...] = rowsum_scr[...] / N1 + col_sum.reshape(1, 1) / N2


@jax.jit
def _chamfer(fixed_verts, movT_aug, mat_aug):
    grid = N1 // BM
    out = pl.pallas_call(
        _chamfer_kernel,
        grid=(grid,),
        in_specs=[
            pl.BlockSpec((BM, 3), lambda i: (i, 0)),
            pl.BlockSpec((4, N2), lambda i: (0, 0)),
            pl.BlockSpec((3, 4), lambda i: (0, 0)),
        ],
        out_specs=pl.BlockSpec((1, 1), lambda i: (0, 0)),
        out_shape=jax.ShapeDtypeStruct((1, 1), jnp.float32),
        scratch_shapes=[
            pltpu.VMEM((8, N2), jnp.float32),
            pltpu.VMEM((1, N2), jnp.float32),
            pltpu.VMEM((1, 1), jnp.float32),
        ],
    )(fixed_verts, movT_aug, mat_aug)
    return out[0, 0]


def kernel(fixed_verts, moving_verts, mat, trans):
    movT_aug = jnp.concatenate(
        [moving_verts.T, jnp.ones((1, N2), jnp.float32)], axis=0)
    mat_aug = jnp.concatenate([mat[0].T, trans[0]], axis=1)
    return _chamfer(fixed_verts, movT_aug, mat_aug)


# speedup vs baseline: 1.7475x; 1.0669x over previous
import jax
import jax.numpy as jnp
from jax.experimental import pallas as pl
from jax.experimental.pallas import tpu as pltpu

N1 = 8192
N2 = 8192
BM = 1024
CW = 2048


def _chamfer_kernel(fixed_ref, movT_ref, mataug_ref, out_ref,
                    rhs_scr, colmin_scr, rowsum_scr):
    i = pl.program_id(0)
    nsteps = pl.num_programs(0)

    @pl.when(i == 0)
    def _init():
        yT = jnp.dot(mataug_ref[...], movT_ref[...],
                     preferred_element_type=jnp.float32)
        rhs_scr[0:3, :] = yT
        rhs_scr[3:4, :] = jnp.sum(yT * yT, axis=0, keepdims=True)
        rhs_scr[4:5, :] = jnp.ones((1, N2), jnp.float32)
        colmin_scr[...] = jnp.full_like(colmin_scr, jnp.inf)
        rowsum_scr[...] = jnp.zeros_like(rowsum_scr)

    xb = fixed_ref[...]
    x2 = jnp.sum(xb * xb, axis=1, keepdims=True)
    lhs = jnp.concatenate(
        [xb * -2.0, jnp.ones((BM, 1), jnp.float32), x2], axis=1)
    row_min = None
    for c in range(N2 // CW):
        d2 = jnp.dot(lhs, rhs_scr[0:5, c * CW:(c + 1) * CW],
                     preferred_element_type=jnp.float32)
        rm = jnp.min(d2, axis=1)
        row_min = rm if row_min is None else jnp.minimum(row_min, rm)
        col_min = jnp.min(d2, axis=0, keepdims=True)
        colmin_scr[0:1, c * CW:(c + 1) * CW] = jnp.minimum(
            colmin_scr[0:1, c * CW:(c + 1) * CW], col_min)

    row_min = jnp.maximum(row_min, 0.0)
    rowsum_scr[...] += jnp.sum(row_min).reshape(1, 1)

    @pl.when(i == nsteps - 1)
    def _fin():
        col_sum = jnp.sum(jnp.maximum(colmin_scr[...], 0.0))
        out_ref[...] = rowsum_scr[...] / N1 + col_sum.reshape(1, 1) / N2


@jax.jit
def _chamfer(fixed_verts, movT_aug, mat_aug):
    grid = N1 // BM
    out = pl.pallas_call(
        _chamfer_kernel,
        grid=(grid,),
        in_specs=[
            pl.BlockSpec((BM, 3), lambda i: (i, 0)),
            pl.BlockSpec((4, N2), lambda i: (0, 0)),
            pl.BlockSpec((3, 4), lambda i: (0, 0)),
        ],
        out_specs=pl.BlockSpec((1, 1), lambda i: (0, 0)),
        out_shape=jax.ShapeDtypeStruct((1, 1), jnp.float32),
        scratch_shapes=[
            pltpu.VMEM((8, N2), jnp.float32),
            pltpu.VMEM((1, N2), jnp.float32),
            pltpu.VMEM((1, 1), jnp.float32),
        ],
    )(fixed_verts, movT_aug, mat_aug)
    return out[0, 0]


def kernel(fixed_verts, moving_verts, mat, trans):
    movT_aug = jnp.concatenate(
        [moving_verts.T, jnp.ones((1, N2), jnp.float32)], axis=0)
    mat_aug = jnp.concatenate([mat[0].T, trans[0]], axis=1)
    return _chamfer(fixed_verts, movT_aug, mat_aug)
